# baseline (speedup 1.0000x reference)
import functools

import jax
import jax.numpy as jnp
import numpy as np
from jax import lax
from jax.experimental import pallas as pl
from jax.experimental.pallas import tpu as pltpu
from jax.experimental.pallas import tpu_sc as plsc

C = 256
N = 10000
K = 16
OUT = 256
LANES = 16
NP = 10240
NB = 512
BN = 4


def _proj_body(x_ref, w_ref, b_ref, t1_ref, t2_ref):
    y = lax.dot_general(
        x_ref[...], w_ref[...], (((0,), (0,)), ((), ())),
        preferred_element_type=jnp.float32,
        precision=lax.Precision.HIGHEST,
    )
    y = y + b_ref[...]
    lo = lax.bitcast_convert_type(y[:, : 2 * OUT].astype(jnp.bfloat16),
                                  jnp.uint16).astype(jnp.int32)
    hi = lax.bitcast_convert_type(y[:, 2 * OUT:].astype(jnp.bfloat16),
                                  jnp.uint16).astype(jnp.int32)
    word = lo | (hi << 16)
    t1_ref[...] = word[:, :OUT]
    t2_ref[...] = word[:, OUT:]


def _project(x2p, wcat, bias):
    return pl.pallas_call(
        _proj_body,
        grid=(NP // NB,),
        in_specs=[
            pl.BlockSpec((C, NB), lambda i: (0, i)),
            pl.BlockSpec((C, 4 * OUT), lambda i: (0, 0)),
            pl.BlockSpec((1, 4 * OUT), lambda i: (0, 0)),
        ],
        out_specs=[
            pl.BlockSpec((NB, OUT), lambda i: (i, 0)),
            pl.BlockSpec((NB, OUT), lambda i: (i, 0)),
        ],
        out_shape=[
            jax.ShapeDtypeStruct((NP, OUT), jnp.int32),
            jax.ShapeDtypeStruct((NP, OUT), jnp.int32),
        ],
    )(x2p, wcat, bias)


def _sc_edge_attn(t1, t2, i1p, i0p):
    info = plsc.get_sparse_core_info()
    nc, ns = info.num_cores, info.num_subcores
    nw = nc * ns
    per_w = NP // nw
    mesh = plsc.VectorSubcoreMesh(core_axis_name="c", subcore_axis_name="s")

    @functools.partial(
        pl.kernel,
        mesh=mesh,
        compiler_params=pltpu.CompilerParams(needs_layout_passes=False),
        out_type=jax.ShapeDtypeStruct((NP, OUT), jnp.float32),
        scratch_types=[
            pltpu.VMEM((per_w * K,), jnp.int32),
            pltpu.VMEM((per_w * K,), jnp.int32),
            pltpu.VMEM((BN * K, OUT), jnp.int32),
            pltpu.VMEM((BN * K, OUT), jnp.int32),
            pltpu.VMEM((BN * K, OUT), jnp.int32),
            pltpu.VMEM((BN * K, OUT), jnp.int32),
            pltpu.VMEM((BN, OUT), jnp.float32),
            pltpu.VMEM((BN, OUT), jnp.float32),
            pltpu.SemaphoreType.DMA,
            pltpu.SemaphoreType.DMA,
            pltpu.SemaphoreType.DMA,
            pltpu.SemaphoreType.DMA,
        ],
    )
    def body(t1_hbm, t2_hbm, i1_hbm, i0_hbm, out_hbm,
             i1_v, i0_v, g1a, g2a, g1b, g2b, out_ra, out_rb,
             sem_a, sem_b, sem_oa, sem_ob):
        wid = lax.axis_index("s") * nc + lax.axis_index("c")
        base = wid * per_w
        pltpu.sync_copy(i1_hbm.at[pl.ds(base * K, per_w * K)], i1_v)
        pltpu.sync_copy(i0_hbm.at[pl.ds(base * K, per_w * K)], i0_v)

        def start(q, g1, g2, sem):
            pltpu.async_copy(t1_hbm.at[i1_v.at[pl.ds(q * BN * K, BN * K)]],
                             g1, sem)
            pltpu.async_copy(t2_hbm.at[i0_v.at[pl.ds(q * BN * K, BN * K)]],
                             g2, sem)

        def wait(q, g1, g2, sem):
            pltpu.make_async_copy(
                t1_hbm.at[i1_v.at[pl.ds(q * BN * K, BN * K)]], g1, sem).wait()
            pltpu.make_async_copy(
                t2_hbm.at[i0_v.at[pl.ds(q * BN * K, BN * K)]], g2, sem).wait()

        def compute(g1, g2, out_r):
            return
            ilv = plsc.PackFormat.INTERLEAVED

            def pksum(k, off, j):
                c1 = plsc.bitcast(g1[k, pl.ds(off + j * LANES, LANES)],
                                  jnp.bfloat16)
                c2 = plsc.bitcast(g2[k, pl.ds(off + j * LANES, LANES)],
                                  jnp.bfloat16)
                return plsc.unpack(c1 + c2, format=ilv)

            for m in range(BN):
                def j_body(j, carry, m=m):
                    slo = jnp.zeros((LANES,), jnp.float32)
                    shi = jnp.zeros((LANES,), jnp.float32)
                    acclo = jnp.zeros((LANES,), jnp.float32)
                    acchi = jnp.zeros((LANES,), jnp.float32)
                    for k in range(K):
                        elo, ehi = pksum(m * K + k, 0, j)
                        alo, ahi = pksum(m * K + k, OUT // 2, j)
                        plo = jnp.exp(alo)
                        phi = jnp.exp(ahi)
                        slo = slo + plo
                        shi = shi + phi
                        acclo = acclo + plo * elo
                        acchi = acchi + phi * ehi
                    out_r[m, pl.ds(j * 32, LANES)] = acclo / slo
                    out_r[m, pl.ds(j * 32 + LANES, LANES)] = acchi / shi
                    return carry
                lax.fori_loop(0, OUT // 32, j_body, 0)

        def out_rows(q):
            return out_hbm.at[pl.ds(base + q * BN, BN), :]

        def wait_store(out_r, q, sem):
            pltpu.make_async_copy(out_r, out_rows(q), sem).wait()

        nq = per_w // BN
        start(0, g1a, g2a, sem_a)

        def loop_body(i2, carry):
            q0 = i2 * 2
            q1 = q0 + 1
            start(q1, g1b, g2b, sem_b)
            wait(q0, g1a, g2a, sem_a)

            @pl.when(i2 > 0)
            def _():
                wait_store(out_ra, q0 - 2, sem_oa)

            compute(g1a, g2a, out_ra)
            pltpu.async_copy(out_ra, out_rows(q0), sem_oa)
            q2 = jnp.minimum(q0 + 2, nq - 1)
            start(q2, g1a, g2a, sem_a)
            wait(q1, g1b, g2b, sem_b)

            @pl.when(i2 > 0)
            def _():
                wait_store(out_rb, q1 - 2, sem_ob)

            compute(g1b, g2b, out_rb)
            pltpu.async_copy(out_rb, out_rows(q1), sem_ob)
            return carry

        lax.fori_loop(0, nq // 2, loop_body, 0)
        wait(nq - 1, g1a, g2a, sem_a)
        wait_store(out_ra, nq - 2, sem_oa)
        wait_store(out_rb, nq - 1, sem_ob)

    return body(t1, t2, i1p, i0p)


def kernel(x, edge_index, We, be, Wa, ba):
    x2 = x[0, :, :, 0]
    x2p = jnp.pad(x2, ((0, 0), (0, NP - N)))
    i1p = jnp.pad(edge_index[1, 0], ((0, NP - N), (0, 0))).reshape(-1)
    i0p = jnp.pad(edge_index[0, 0], ((0, NP - N), (0, 0))).reshape(-1)
    we_i, we_d = We[:, :C], We[:, C:]
    wa_i, wa_d = Wa[:, :C], Wa[:, C:]
    wcat = jnp.concatenate(
        [(we_i - we_d).T, (wa_i - wa_d).T, we_d.T, wa_d.T], axis=1)
    bias = jnp.concatenate(
        [be, jnp.zeros((3 * OUT,), jnp.float32)])[None, :]
    w = np.arange(OUT)
    j, t = w // LANES, w % LANES
    lo_idx = np.concatenate([tb * 2 * OUT + 32 * j + t for tb in (0, 1)])
    hi_idx = np.concatenate([tb * 2 * OUT + 32 * j + LANES + t for tb in (0, 1)])
    perm = np.concatenate([lo_idx, hi_idx])
    wcat = wcat[:, perm]
    bias = bias[:, perm]
    t1, t2 = _project(x2p, wcat, bias)
    rows = _sc_edge_attn(t1, t2, i1p, i0p)
    return rows[:N].T[None, :, :, None]


# speedup vs baseline: 1.0247x; 1.0247x over previous
import functools

import jax
import jax.numpy as jnp
import numpy as np
from jax import lax
from jax.experimental import pallas as pl
from jax.experimental.pallas import tpu as pltpu
from jax.experimental.pallas import tpu_sc as plsc

C = 256
N = 10000
K = 16
OUT = 256
LANES = 16
NP = 10240
NB = 512
BN = 4


def _proj_body(x_ref, w_ref, b_ref, t1_ref, t2_ref):
    y = lax.dot_general(
        x_ref[...], w_ref[...], (((0,), (0,)), ((), ())),
        preferred_element_type=jnp.float32,
        precision=lax.Precision.DEFAULT,
    )
    y = y + b_ref[...]
    lo = lax.bitcast_convert_type(y[:, : 2 * OUT].astype(jnp.bfloat16),
                                  jnp.uint16).astype(jnp.int32)
    hi = lax.bitcast_convert_type(y[:, 2 * OUT:].astype(jnp.bfloat16),
                                  jnp.uint16).astype(jnp.int32)
    word = lo | (hi << 16)
    t1_ref[...] = word[:, :OUT]
    t2_ref[...] = word[:, OUT:]


def _project(x2p, wcat, bias):
    return pl.pallas_call(
        _proj_body,
        grid=(NP // NB,),
        in_specs=[
            pl.BlockSpec((C, NB), lambda i: (0, i)),
            pl.BlockSpec((C, 4 * OUT), lambda i: (0, 0)),
            pl.BlockSpec((1, 4 * OUT), lambda i: (0, 0)),
        ],
        out_specs=[
            pl.BlockSpec((NB, OUT), lambda i: (i, 0)),
            pl.BlockSpec((NB, OUT), lambda i: (i, 0)),
        ],
        out_shape=[
            jax.ShapeDtypeStruct((NP, OUT), jnp.int32),
            jax.ShapeDtypeStruct((NP, OUT), jnp.int32),
        ],
    )(x2p, wcat, bias)


def _tr_body(in_ref, out_ref):
    out_ref[...] = in_ref[...].T


def _transpose(rows):
    return pl.pallas_call(
        _tr_body,
        grid=(NP // NB,),
        in_specs=[pl.BlockSpec((NB, OUT), lambda i: (i, 0))],
        out_specs=pl.BlockSpec((OUT, NB), lambda i: (0, i)),
        out_shape=jax.ShapeDtypeStruct((OUT, N), jnp.float32),
    )(rows)


def _sc_edge_attn(t1, t2, i1p, i0p):
    info = plsc.get_sparse_core_info()
    nc, ns = info.num_cores, info.num_subcores
    nw = nc * ns
    per_w = NP // nw
    mesh = plsc.VectorSubcoreMesh(core_axis_name="c", subcore_axis_name="s")

    @functools.partial(
        pl.kernel,
        mesh=mesh,
        compiler_params=pltpu.CompilerParams(needs_layout_passes=False),
        out_type=jax.ShapeDtypeStruct((NP, OUT), jnp.float32),
        scratch_types=[
            pltpu.VMEM((per_w * K,), jnp.int32),
            pltpu.VMEM((per_w * K,), jnp.int32),
            pltpu.VMEM((BN * K, OUT), jnp.int32),
            pltpu.VMEM((BN * K, OUT), jnp.int32),
            pltpu.VMEM((BN * K, OUT), jnp.int32),
            pltpu.VMEM((BN * K, OUT), jnp.int32),
            pltpu.VMEM((BN, OUT), jnp.float32),
            pltpu.VMEM((BN, OUT), jnp.float32),
            pltpu.SemaphoreType.DMA,
            pltpu.SemaphoreType.DMA,
            pltpu.SemaphoreType.DMA,
            pltpu.SemaphoreType.DMA,
        ],
    )
    def body(t1_hbm, t2_hbm, i1_hbm, i0_hbm, out_hbm,
             i1_v, i0_v, g1a, g2a, g1b, g2b, out_ra, out_rb,
             sem_a, sem_b, sem_oa, sem_ob):
        wid = lax.axis_index("s") * nc + lax.axis_index("c")
        base = wid * per_w
        pltpu.sync_copy(i1_hbm.at[pl.ds(base * K, per_w * K)], i1_v)
        pltpu.sync_copy(i0_hbm.at[pl.ds(base * K, per_w * K)], i0_v)

        def start(q, g1, g2, sem):
            pltpu.async_copy(t1_hbm.at[i1_v.at[pl.ds(q * BN * K, BN * K)]],
                             g1, sem)
            pltpu.async_copy(t2_hbm.at[i0_v.at[pl.ds(q * BN * K, BN * K)]],
                             g2, sem)

        def wait(q, g1, g2, sem):
            pltpu.make_async_copy(
                t1_hbm.at[i1_v.at[pl.ds(q * BN * K, BN * K)]], g1, sem).wait()
            pltpu.make_async_copy(
                t2_hbm.at[i0_v.at[pl.ds(q * BN * K, BN * K)]], g2, sem).wait()

        def compute(g1, g2, out_r):
            ilv = plsc.PackFormat.INTERLEAVED

            def pksum(k, off, j):
                c1 = plsc.bitcast(g1[k, pl.ds(off + j * LANES, LANES)],
                                  jnp.bfloat16)
                c2 = plsc.bitcast(g2[k, pl.ds(off + j * LANES, LANES)],
                                  jnp.bfloat16)
                return plsc.unpack(c1 + c2, format=ilv)

            for m in range(BN):
                def j_body(j, carry, m=m):
                    slo = jnp.zeros((LANES,), jnp.float32)
                    shi = jnp.zeros((LANES,), jnp.float32)
                    acclo = jnp.zeros((LANES,), jnp.float32)
                    acchi = jnp.zeros((LANES,), jnp.float32)
                    for k in range(K):
                        elo, ehi = pksum(m * K + k, 0, j)
                        alo, ahi = pksum(m * K + k, OUT // 2, j)
                        plo = jnp.exp(alo)
                        phi = jnp.exp(ahi)
                        slo = slo + plo
                        shi = shi + phi
                        acclo = acclo + plo * elo
                        acchi = acchi + phi * ehi
                    out_r[m, pl.ds(j * 32, LANES)] = acclo / slo
                    out_r[m, pl.ds(j * 32 + LANES, LANES)] = acchi / shi
                    return carry
                lax.fori_loop(0, OUT // 32, j_body, 0)

        def out_rows(q):
            return out_hbm.at[pl.ds(base + q * BN, BN), :]

        def wait_store(out_r, q, sem):
            pltpu.make_async_copy(out_r, out_rows(q), sem).wait()

        nq = per_w // BN
        start(0, g1a, g2a, sem_a)

        def loop_body(i2, carry):
            q0 = i2 * 2
            q1 = q0 + 1
            start(q1, g1b, g2b, sem_b)
            wait(q0, g1a, g2a, sem_a)

            @pl.when(i2 > 0)
            def _():
                wait_store(out_ra, q0 - 2, sem_oa)

            compute(g1a, g2a, out_ra)
            pltpu.async_copy(out_ra, out_rows(q0), sem_oa)
            q2 = jnp.minimum(q0 + 2, nq - 1)
            start(q2, g1a, g2a, sem_a)
            wait(q1, g1b, g2b, sem_b)

            @pl.when(i2 > 0)
            def _():
                wait_store(out_rb, q1 - 2, sem_ob)

            compute(g1b, g2b, out_rb)
            pltpu.async_copy(out_rb, out_rows(q1), sem_ob)
            return carry

        lax.fori_loop(0, nq // 2, loop_body, 0)
        wait(nq - 1, g1a, g2a, sem_a)
        wait_store(out_ra, nq - 2, sem_oa)
        wait_store(out_rb, nq - 1, sem_ob)

    return body(t1, t2, i1p, i0p)


def kernel(x, edge_index, We, be, Wa, ba):
    x2 = x[0, :, :, 0]
    i1p = jnp.pad(edge_index[1, 0], ((0, NP - N), (0, 0))).reshape(-1)
    i0p = jnp.pad(edge_index[0, 0], ((0, NP - N), (0, 0))).reshape(-1)
    we_i, we_d = We[:, :C], We[:, C:]
    wa_i, wa_d = Wa[:, :C], Wa[:, C:]
    wcat = jnp.concatenate(
        [(we_i - we_d).T, (wa_i - wa_d).T, we_d.T, wa_d.T], axis=1)
    bias = jnp.concatenate(
        [be, jnp.zeros((3 * OUT,), jnp.float32)])[None, :]
    w = np.arange(OUT)
    j, t = w // LANES, w % LANES
    lo_idx = np.concatenate([tb * 2 * OUT + 32 * j + t for tb in (0, 1)])
    hi_idx = np.concatenate([tb * 2 * OUT + 32 * j + LANES + t for tb in (0, 1)])
    perm = np.concatenate([lo_idx, hi_idx])
    wcat = wcat[:, perm]
    bias = bias[:, perm]
    t1, t2 = _project(x2, wcat, bias)
    rows = _sc_edge_attn(t1, t2, i1p, i0p)
    return _transpose(rows)[None, :, :, None]
